# baseline (speedup 1.0000x reference)
import jax
import jax.numpy as jnp
from jax import lax
from jax.experimental import pallas as pl
from jax.experimental.pallas import tpu as pltpu
from jax.experimental.pallas import tpu_sc as plsc

B, L, V, D = 1024, 200, 6, 128
NC, NS, LANES = 2, 16, 16
NW = NC * NS
NCHUNK = 4
BCH = B // NCHUNK
BPW = BCH // NW
NVEC = (L + LANES - 1) // LANES
TAIL = L - (NVEC - 1) * LANES

ROW_STRIDE = D + 1
REP_STRIDE = ((V * ROW_STRIDE + LANES) // LANES) * LANES + 1
TBL_WORDS = LANES * REP_STRIDE


def _body(seq_hbm, tbl_hbm, out_hbm, seq_v, tbl_v, blk_v, sems):
    wid = lax.axis_index("s") * NC + lax.axis_index("c")
    base_b = wid * BPW
    pltpu.sync_copy(seq_hbm.at[pl.ds(base_b * L, BPW * L)], seq_v)
    pltpu.sync_copy(tbl_hbm, tbl_v)

    lanes = jnp.arange(LANES, dtype=jnp.int32)
    lane_base = lanes * REP_STRIDE
    tail_mask = lanes < TAIL
    tail_li = jnp.minimum((NVEC - 1) * LANES + lanes, L - 1)

    def per_b(bi, carry):
        buf = lax.rem(bi, 2)

        @pl.when(bi >= 2)
        def _wait():
            pltpu.make_async_copy(
                blk_v.at[buf], out_hbm.at[base_b + bi - 2], sems.at[buf]
            ).wait()

        bufv = jnp.full((LANES,), buf, jnp.int32)
        for j in range(NVEC):
            li = tail_li if j == NVEC - 1 else j * LANES + lanes
            seqv = plsc.load_gather(seq_v, [bi * L + li])
            addr0 = lane_base + seqv * ROW_STRIDE
            addrs0 = tuple(addr0 + k for k in range(16))
            if j < NVEC - 1:
                @plsc.parallel_loop(0, D, step=16, carry=addrs0)
                def _dloop(d0, addrs):
                    for k in range(16):
                        val = plsc.load_gather(tbl_v, [addrs[k]])
                        blk_v[buf, d0 + k, pl.ds(j * LANES, LANES)] = val
                    return tuple(a + 16 for a in addrs)
            else:
                @plsc.parallel_loop(0, D, step=16, carry=addrs0)
                def _dtail(d0, addrs):
                    for k in range(16):
                        val = plsc.load_gather(tbl_v, [addrs[k]])
                        plsc.store_scatter(
                            blk_v,
                            [bufv, jnp.full((LANES,), d0 + k, jnp.int32), tail_li],
                            val,
                            mask=tail_mask,
                        )
                    return tuple(a + 16 for a in addrs)
        pltpu.async_copy(blk_v.at[buf], out_hbm.at[base_b + bi], sems.at[buf])
        return carry

    lax.fori_loop(0, BPW, per_b, jnp.int32(0))
    for t in (BPW - 2, BPW - 1):
        pltpu.make_async_copy(
            blk_v.at[t % 2], out_hbm.at[base_b + t], sems.at[t % 2]
        ).wait()


@jax.jit
def kernel(seq, table):
    seq_flat = seq.astype(jnp.int32).reshape(B * L)
    row_pad = jnp.pad(table, ((0, 0), (0, ROW_STRIDE - D))).reshape(-1)
    rep = jnp.pad(row_pad, (0, REP_STRIDE - row_pad.shape[0]))
    tbl_flat = jnp.tile(rep, LANES)
    run = pl.kernel(
        _body,
        out_type=jax.ShapeDtypeStruct((BCH, D, L), jnp.float32),
        mesh=plsc.VectorSubcoreMesh(core_axis_name="c", subcore_axis_name="s"),
        compiler_params=pltpu.CompilerParams(needs_layout_passes=False),
        scratch_types=[
            pltpu.VMEM((BPW * L,), jnp.int32),
            pltpu.VMEM((TBL_WORDS,), jnp.float32),
            pltpu.VMEM((2, D, L), jnp.float32),
            pltpu.SemaphoreType.DMA((2,)),
        ],
    )
    outs = [
        run(lax.dynamic_slice(seq_flat, (c * BCH * L,), (BCH * L,)), tbl_flat)
        for c in range(NCHUNK)
    ]
    out = jnp.empty((B, D, L), jnp.float32)
    for c in range(NCHUNK):
        out = lax.dynamic_update_slice(out, outs[c], (c * BCH, 0, 0))
    return out


# speedup vs baseline: 1.4571x; 1.4571x over previous
import jax
import jax.numpy as jnp
from jax import lax
from jax.experimental import pallas as pl
from jax.experimental.pallas import tpu as pltpu
from jax.experimental.pallas import tpu_sc as plsc

B, L, V, D = 1024, 200, 6, 128
NC, NS, LANES = 2, 16, 16
NW = NC * NS
NCHUNK = 1
BCH = B // NCHUNK
BPW = BCH // NW
LP = 256
NVEC = (L + LANES - 1) // LANES
TAIL = L - (NVEC - 1) * LANES

ROW_STRIDE = D + 1
REP_STRIDE = ((V * ROW_STRIDE + LANES) // LANES) * LANES + 1
TBL_WORDS = LANES * REP_STRIDE


def _body(seq_hbm, tbl_hbm, out_hbm, seq_v, tbl_v, blk_v, sems):
    wid = lax.axis_index("s") * NC + lax.axis_index("c")
    base_b = wid * BPW
    pltpu.sync_copy(seq_hbm.at[pl.ds(base_b * L, BPW * L)], seq_v)
    pltpu.sync_copy(tbl_hbm, tbl_v)

    lanes = jnp.arange(LANES, dtype=jnp.int32)
    lane_base = lanes * REP_STRIDE
    tail_mask = lanes < TAIL
    tail_li = jnp.minimum((NVEC - 1) * LANES + lanes, L - 1)

    def per_b(bi, carry):
        buf = lax.rem(bi, 2)

        @pl.when(bi >= 2)
        def _wait():
            pltpu.make_async_copy(
                blk_v.at[buf], out_hbm.at[base_b + bi - 2], sems.at[buf]
            ).wait()

        for j in range(NVEC):
            li = tail_li if j == NVEC - 1 else j * LANES + lanes
            seqv = plsc.load_gather(seq_v, [bi * L + li])
            addr0 = lane_base + seqv * ROW_STRIDE
            addrs0 = tuple(addr0 + k for k in range(16))
            jc = (j * LANES) // 128
            off = (j * LANES) % 128

            @plsc.parallel_loop(0, D, step=16, carry=addrs0)
            def _dloop(d0, addrs):
                row = (d0 // 8) * 2
                for k in range(16):
                    val = plsc.load_gather(tbl_v, [addrs[k]])
                    blk_v[buf, row + (k // 8) * 2 + jc, k % 8, pl.ds(off, LANES)] = val
                return tuple(a + 16 for a in addrs)
        pltpu.async_copy(blk_v.at[buf], out_hbm.at[base_b + bi], sems.at[buf])
        return carry

    lax.fori_loop(0, BPW, per_b, jnp.int32(0))
    for t in (BPW - 2, BPW - 1):
        pltpu.make_async_copy(
            blk_v.at[t % 2], out_hbm.at[base_b + t], sems.at[t % 2]
        ).wait()


@jax.jit
def kernel(seq, table):
    seq_flat = seq.astype(jnp.int32).reshape(B * L)
    row_pad = jnp.pad(table, ((0, 0), (0, ROW_STRIDE - D))).reshape(-1)
    rep = jnp.pad(row_pad, (0, REP_STRIDE - row_pad.shape[0]))
    tbl_flat = jnp.tile(rep, LANES)
    run = pl.kernel(
        _body,
        out_type=jax.ShapeDtypeStruct((BCH, (D // 8) * (LP // 128), 8, 128), jnp.float32),
        mesh=plsc.VectorSubcoreMesh(core_axis_name="c", subcore_axis_name="s"),
        compiler_params=pltpu.CompilerParams(needs_layout_passes=False),
        scratch_types=[
            pltpu.VMEM((BPW * L,), jnp.int32),
            pltpu.VMEM((TBL_WORDS,), jnp.float32),
            pltpu.VMEM((2, (D // 8) * (LP // 128), 8, 128), jnp.float32),
            pltpu.SemaphoreType.DMA((2,)),
        ],
    )
    out_t = run(seq_flat, tbl_flat)
    out = out_t.reshape(B, D // 8, LP // 128, 8, 128)
    out = out.transpose(0, 1, 3, 2, 4).reshape(B, D, LP)
    return out[:, :, :L]
